# baseline (speedup 1.0000x reference)
import functools

import jax
import jax.numpy as jnp
from jax import lax
from jax.experimental import pallas as pl
from jax.experimental.pallas import tpu as pltpu
from jax.experimental.pallas import tpu_sc as plsc

VOCAB = 50257
HIDDEN = 768
NC = 2
NS = 16
NW = NC * NS
CH = 64

_mesh = plsc.VectorSubcoreMesh(core_axis_name="c", subcore_axis_name="s")


def _make_gather(n_total: int):
  assert n_total % NW == 0
  bpw = n_total // NW
  assert bpw % CH == 0
  nch = bpw // CH

  @functools.partial(
      pl.kernel,
      mesh=_mesh,
      out_type=jax.ShapeDtypeStruct((NW, nch, CH, HIDDEN), jnp.float32),
      scratch_types=[
          pltpu.VMEM((nch, CH), jnp.int32),
          pltpu.VMEM((2, CH, HIDDEN), jnp.float32),
          pltpu.SemaphoreType.DMA,
          pltpu.SemaphoreType.DMA,
          pltpu.SemaphoreType.DMA,
          pltpu.SemaphoreType.DMA,
      ],
  )
  def gather_kernel(table_hbm, ids_hbm, out_hbm, idx_v, rows_v,
                    gsem0, gsem1, osem0, osem1):
    wid = lax.axis_index("s") * NC + lax.axis_index("c")
    pltpu.sync_copy(ids_hbm.at[wid], idx_v)

    gsems = (gsem0, gsem1)
    osems = (osem0, osem1)
    cps = [None] * nch
    ocs = [None] * nch
    cps[0] = pltpu.async_copy(table_hbm.at[idx_v.at[0]], rows_v.at[0], gsems[0])
    for g in range(nch):
      b = g & 1
      cps[g].wait()
      ocs[g] = pltpu.async_copy(rows_v.at[b], out_hbm.at[wid, g], osems[b])
      if g + 1 < nch:
        if g >= 1:
          ocs[g - 1].wait()
        cps[g + 1] = pltpu.async_copy(
            table_hbm.at[idx_v.at[g + 1]], rows_v.at[1 - b], gsems[1 - b])
    ocs[nch - 2].wait()
    ocs[nch - 1].wait()

  return gather_kernel, bpw, nch


def kernel(input_ids, word_embeddings):
  b, s = input_ids.shape
  n = b * s
  gather, bpw, nch = _make_gather(n)
  ids = input_ids.reshape(NW, nch, CH).astype(jnp.int32)
  out = gather(word_embeddings, ids)
  return out.reshape(b, s, HIDDEN)


# speedup vs baseline: 1.0182x; 1.0182x over previous
import functools

import jax
import jax.numpy as jnp
from jax import lax
from jax.experimental import pallas as pl
from jax.experimental.pallas import tpu as pltpu
from jax.experimental.pallas import tpu_sc as plsc

VOCAB = 50257
HIDDEN = 768
NC = 2
NS = 16
NW = NC * NS
CH = 32
NBUF = 4

_mesh = plsc.VectorSubcoreMesh(core_axis_name="c", subcore_axis_name="s")


def _make_gather(n_total: int):
  assert n_total % NW == 0
  bpw = n_total // NW
  assert bpw % CH == 0
  nch = bpw // CH

  @functools.partial(
      pl.kernel,
      mesh=_mesh,
      out_type=jax.ShapeDtypeStruct((NW, nch, CH, HIDDEN), jnp.float32),
      scratch_types=[
          pltpu.VMEM((nch, CH), jnp.int32),
          pltpu.VMEM((NBUF, CH, HIDDEN), jnp.float32),
          pltpu.SemaphoreType.DMA,
          pltpu.SemaphoreType.DMA,
          pltpu.SemaphoreType.DMA,
          pltpu.SemaphoreType.DMA,
          pltpu.SemaphoreType.DMA,
          pltpu.SemaphoreType.DMA,
          pltpu.SemaphoreType.DMA,
          pltpu.SemaphoreType.DMA,
      ],
  )
  def gather_kernel(table_hbm, ids_hbm, out_hbm, idx_v, rows_v,
                    g0, g1, g2, g3, o0, o1, o2, o3):
    wid = lax.axis_index("s") * NC + lax.axis_index("c")
    pltpu.sync_copy(ids_hbm.at[wid], idx_v)

    gsems = (g0, g1, g2, g3)
    osems = (o0, o1, o2, o3)
    cps = [None] * nch
    ocs = [None] * nch
    for k in range(3):
      cps[k] = pltpu.async_copy(
          table_hbm.at[idx_v.at[k]], rows_v.at[k], gsems[k])
    for g in range(nch):
      b = g % NBUF
      cps[g].wait()
      ocs[g] = pltpu.async_copy(rows_v.at[b], out_hbm.at[wid, g], osems[b])
      nxt = g + 3
      if nxt < nch:
        if nxt - NBUF >= 0:
          ocs[nxt - NBUF].wait()
        cps[nxt] = pltpu.async_copy(
            table_hbm.at[idx_v.at[nxt]], rows_v.at[nxt % NBUF], gsems[nxt % NBUF])
    for g in range(nch - NBUF, nch):
      ocs[g].wait()

  return gather_kernel, bpw, nch


def kernel(input_ids, word_embeddings):
  b, s = input_ids.shape
  n = b * s
  gather, bpw, nch = _make_gather(n)
  ids = input_ids.reshape(NW, nch, CH).astype(jnp.int32)
  out = gather(word_embeddings, ids)
  return out.reshape(b, s, HIDDEN)


# speedup vs baseline: 1.0219x; 1.0036x over previous
import functools

import jax
import jax.numpy as jnp
from jax import lax
from jax.experimental import pallas as pl
from jax.experimental.pallas import tpu as pltpu
from jax.experimental.pallas import tpu_sc as plsc

VOCAB = 50257
HIDDEN = 768
NC = 2
NS = 16
NW = NC * NS
CH = 32
NBUF = 5
DEPTH = 4

_mesh = plsc.VectorSubcoreMesh(core_axis_name="c", subcore_axis_name="s")


def _make_gather(n_total: int):
  assert n_total % NW == 0
  bpw = n_total // NW
  assert bpw % CH == 0
  nch = bpw // CH

  @functools.partial(
      pl.kernel,
      mesh=_mesh,
      out_type=jax.ShapeDtypeStruct((NW, nch, CH, HIDDEN), jnp.float32),
      scratch_types=[
          pltpu.VMEM((nch, CH), jnp.int32),
          pltpu.VMEM((NBUF, CH, HIDDEN), jnp.float32),
          *([pltpu.SemaphoreType.DMA] * 10),
      ],
  )
  def gather_kernel(table_hbm, ids_hbm, out_hbm, idx_v, rows_v, *sems):
    wid = lax.axis_index("s") * NC + lax.axis_index("c")
    pltpu.sync_copy(ids_hbm.at[wid], idx_v)

    gsems = sems[:NBUF]
    osems = sems[NBUF:]
    cps = [None] * nch
    ocs = [None] * nch
    for k in range(DEPTH):
      cps[k] = pltpu.async_copy(
          table_hbm.at[idx_v.at[k]], rows_v.at[k], gsems[k])
    for g in range(nch):
      b = g % NBUF
      cps[g].wait()
      ocs[g] = pltpu.async_copy(rows_v.at[b], out_hbm.at[wid, g], osems[b])
      nxt = g + DEPTH
      if nxt < nch:
        if nxt - NBUF >= 0:
          ocs[nxt - NBUF].wait()
        cps[nxt] = pltpu.async_copy(
            table_hbm.at[idx_v.at[nxt]], rows_v.at[nxt % NBUF],
            gsems[nxt % NBUF])
    for g in range(nch - NBUF, nch):
      ocs[g].wait()

  return gather_kernel, bpw, nch


def kernel(input_ids, word_embeddings):
  b, s = input_ids.shape
  n = b * s
  gather, bpw, nch = _make_gather(n)
  ids = input_ids.reshape(NW, nch, CH).astype(jnp.int32)
  out = gather(word_embeddings, ids)
  return out.reshape(b, s, HIDDEN)
